# Initial kernel scaffold; baseline (speedup 1.0000x reference)
import jax
import jax.numpy as jnp
from jax.experimental import pallas as pl


def kernel(query, doc, boundaries, emb_table):
    raise NotImplementedError("write your pallas kernel here")


# speedup vs baseline: 11.8994x; 11.8994x over previous
import functools

import jax
import jax.numpy as jnp
from jax import lax
from jax.experimental import pallas as pl
from jax.experimental.pallas import tpu as pltpu
from jax.experimental.pallas import tpu_sc as plsc

B = 16384
D = 64
NUM_BOUNDARIES = 100
PAD_BOUND = 128
EMB_DIM = 32

ROWS_PER_BLOCK = 2048
GRID = B // ROWS_PER_BLOCK

_info = plsc.get_sparse_core_info()
NC = _info.num_cores
NS = _info.num_subcores
NW = NC * NS
B_PER_W = B // NW
CHUNK = 128
NCHUNK = B_PER_W // CHUNK


def _bucket_body(q_ref, d_ref, b_ref, out_ref):
    f = jnp.sum(q_ref[...] * d_ref[...], axis=1)
    cmp = f[:, None] >= b_ref[...][None, :]
    out_ref[...] = jnp.sum(cmp.astype(jnp.int32), axis=1)


def _compute_buckets(query, doc, boundaries_padded):
    return pl.pallas_call(
        _bucket_body,
        grid=(GRID,),
        in_specs=[
            pl.BlockSpec((ROWS_PER_BLOCK, D), lambda i: (i, 0)),
            pl.BlockSpec((ROWS_PER_BLOCK, D), lambda i: (i, 0)),
            pl.BlockSpec((PAD_BOUND,), lambda i: (0,)),
        ],
        out_specs=pl.BlockSpec((ROWS_PER_BLOCK,), lambda i: (i,)),
        out_shape=jax.ShapeDtypeStruct((B,), jnp.int32),
    )(query, doc, boundaries_padded)


VOCAB_PAD = 128


@functools.partial(
    pl.kernel,
    mesh=plsc.VectorSubcoreMesh(core_axis_name="c", subcore_axis_name="s"),
    out_type=jax.ShapeDtypeStruct((B * EMB_DIM,), jnp.float32),
    compiler_params=pltpu.CompilerParams(needs_layout_passes=False),
    scratch_types=[
        pltpu.VMEM((VOCAB_PAD * EMB_DIM,), jnp.float32),
        pltpu.VMEM((B_PER_W,), jnp.int32),
        pltpu.VMEM((B_PER_W * EMB_DIM,), jnp.float32),
    ],
)
def _sc_gather(table_hbm, idx_hbm, out_hbm, table_v, idx_v, rows_v):
    wid = lax.axis_index("s") * NC + lax.axis_index("c")
    base = wid * B_PER_W
    pltpu.sync_copy(idx_hbm.at[pl.ds(base, B_PER_W)], idx_v)
    pltpu.sync_copy(table_hbm, table_v)
    lane = lax.iota(jnp.int32, 16)

    def body(i, carry):
        bucket_vec = idx_v[pl.ds(i * 16, 16)]
        src_base = bucket_vec * EMB_DIM
        dst_base = i * (16 * EMB_DIM) + lane * EMB_DIM
        for c in range(EMB_DIM):
            vals = plsc.load_gather(table_v, [src_base + c])
            plsc.store_scatter(rows_v, [dst_base + c], vals)
        return carry

    lax.fori_loop(0, B_PER_W // 16, body, 0)
    pltpu.sync_copy(rows_v, out_hbm.at[pl.ds(base * EMB_DIM, B_PER_W * EMB_DIM)])


def kernel(query, doc, boundaries, emb_table):
    boundaries_padded = jnp.concatenate(
        [boundaries,
         jnp.full((PAD_BOUND - NUM_BOUNDARIES,), jnp.inf, jnp.float32)])
    bucket = _compute_buckets(query, doc, boundaries_padded)
    table_padded = jnp.zeros((VOCAB_PAD, EMB_DIM), jnp.float32).at[:101].set(emb_table)
    out_flat = _sc_gather(table_padded.reshape(-1), bucket)
    return out_flat.reshape(B, EMB_DIM)
